# Initial kernel scaffold; baseline (speedup 1.0000x reference)
import jax
import jax.numpy as jnp
from jax.experimental import pallas as pl
from jax.experimental.pallas import tpu as pltpu

B = 2
N = 4096
D = 256
BI = 512
BJ = 512
RB = 512


def _knn2_kernel(xi_ref, xj_ref, out_ref):
    i = pl.program_id(1)
    j = pl.program_id(2)
    xi = xi_ref[0]
    xj = xj_ref[0]
    sqi = jnp.sum(xi * xi, axis=1)
    sqj = jnp.sum(xj * xj, axis=1)
    dott = jax.lax.dot_general(
        xj, xi, (((1,), (1,)), ((), ())), preferred_element_type=jnp.float32
    )
    d2t = jnp.maximum(sqj[:, None] + sqi[None, :] - 2.0 * dott, 0.0)
    gcol = jax.lax.broadcasted_iota(jnp.int32, (BJ, BI), 0) + j * BJ
    grow = jax.lax.broadcasted_iota(jnp.int32, (BJ, BI), 1) + i * BI
    inf = jnp.float32(jnp.inf)
    d2t = jnp.where(gcol == grow, inf, d2t)
    m1 = jnp.min(d2t, axis=0, keepdims=True)
    ism = d2t == m1
    minrow = jnp.min(jnp.where(ism, gcol, jnp.int32(2**30)), axis=0, keepdims=True)
    m2 = jnp.min(jnp.where(gcol == minrow, inf, d2t), axis=0, keepdims=True)
    first = j == 0
    r1 = jnp.where(first, inf, out_ref[0, 0:1, :])
    r2 = jnp.where(first, inf, out_ref[0, 1:2, :])
    out_ref[0, 0:1, :] = jnp.minimum(r1, m1)
    out_ref[0, 1:2, :] = jnp.minimum(jnp.maximum(r1, m1), jnp.minimum(r2, m2))


def _twonn_kernel(dfull_ref, dcol_ref, o1_ref, o2_ref):
    i = pl.program_id(1)
    d1f = dfull_ref[0, 0:1, :]
    d2f = dfull_ref[0, 1:2, :]
    tf = 0.5 * (jnp.log(d2f) - jnp.log(d1f))
    d1b = dcol_ref[0, :, 0:1]
    d2b = dcol_ref[0, :, 1:2]
    tb = 0.5 * (jnp.log(d2b) - jnp.log(d1b))
    gcol = jax.lax.broadcasted_iota(jnp.int32, (RB, N), 1)
    grow = jax.lax.broadcasted_iota(jnp.int32, (RB, N), 0) + i * RB
    less = (tf < tb).astype(jnp.float32)
    ties = jnp.logical_and(tf == tb, gcol < grow).astype(jnp.float32)
    rank = jnp.sum(less + ties, axis=1, keepdims=True)
    y = jnp.log(jnp.float32(N)) - jnp.log(jnp.float32(N) - rank)
    sxy = jnp.sum(tb * y)
    sxx = jnp.sum(tb * tb)
    first = i == 0
    o1_ref[...] = jnp.where(first, 0.0, o1_ref[...]) + sxy
    o2_ref[...] = jnp.where(first, 0.0, o2_ref[...]) + sxx


def kernel(X):
    d12 = pl.pallas_call(
        _knn2_kernel,
        grid=(B, N // BI, N // BJ),
        in_specs=[
            pl.BlockSpec((1, BI, D), lambda b, i, j: (b, i, 0)),
            pl.BlockSpec((1, BJ, D), lambda b, i, j: (b, j, 0)),
        ],
        out_specs=pl.BlockSpec((1, 2, BI), lambda b, i, j: (b, 0, i)),
        out_shape=jax.ShapeDtypeStruct((B, 2, N), jnp.float32),
        compiler_params=pltpu.CompilerParams(
            dimension_semantics=("parallel", "parallel", "arbitrary"),
        ),
    )(X, X)
    d12_t = jnp.transpose(d12, (0, 2, 1))
    o1, o2 = pl.pallas_call(
        _twonn_kernel,
        grid=(B, N // RB),
        in_specs=[
            pl.BlockSpec((1, 2, N), lambda b, i: (b, 0, 0)),
            pl.BlockSpec((1, RB, 2), lambda b, i: (b, i, 0)),
        ],
        out_specs=[
            pl.BlockSpec((1, 128), lambda b, i: (b, 0)),
            pl.BlockSpec((1, 128), lambda b, i: (b, 0)),
        ],
        out_shape=[
            jax.ShapeDtypeStruct((B, 128), jnp.float32),
            jax.ShapeDtypeStruct((B, 128), jnp.float32),
        ],
        compiler_params=pltpu.CompilerParams(
            dimension_semantics=("parallel", "arbitrary"),
        ),
    )(d12, d12_t)
    return o1[:, 0] / o2[:, 0]


# speedup vs baseline: 51.6955x; 51.6955x over previous
import jax
import jax.numpy as jnp
from jax.experimental import pallas as pl
from jax.experimental.pallas import tpu as pltpu

B = 2
N = 4096
D = 256
BI = 512
BJ = 512
RB = 512


def _knn2_kernel(xi_ref, xj_ref, out_ref):
    i = pl.program_id(1)
    j = pl.program_id(2)
    xi = xi_ref[0]
    xj = xj_ref[0]
    sqi = jnp.sum(xi * xi, axis=1)
    sqj = jnp.sum(xj * xj, axis=1)
    dott = jax.lax.dot_general(
        xj, xi, (((1,), (1,)), ((), ())), preferred_element_type=jnp.float32
    )
    d2t = jnp.maximum(sqj[:, None] + sqi[None, :] - 2.0 * dott, 0.0)
    gcol = jax.lax.broadcasted_iota(jnp.int32, (BJ, BI), 0) + j * BJ
    grow = jax.lax.broadcasted_iota(jnp.int32, (BJ, BI), 1) + i * BI
    inf = jnp.float32(jnp.inf)
    d2t = jnp.where(gcol == grow, inf, d2t)
    m1 = jnp.min(d2t, axis=0, keepdims=True)
    ism = d2t == m1
    minrow = jnp.min(jnp.where(ism, gcol, jnp.int32(2**30)), axis=0, keepdims=True)
    m2 = jnp.min(jnp.where(gcol == minrow, inf, d2t), axis=0, keepdims=True)
    first = j == 0
    r1 = jnp.where(first, inf, out_ref[0, 0:1, :])
    r2 = jnp.where(first, inf, out_ref[0, 1:2, :])
    out_ref[0, 0:1, :] = jnp.minimum(r1, m1)
    out_ref[0, 1:2, :] = jnp.minimum(jnp.maximum(r1, m1), jnp.minimum(r2, m2))


def _twonn_kernel(dfull_ref, dcol_ref, o1_ref, o2_ref):
    i = pl.program_id(1)
    d1f = dfull_ref[0, 0:1, :]
    d2f = dfull_ref[0, 1:2, :]
    tf = 0.5 * (jnp.log(d2f) - jnp.log(d1f))
    d1b = dcol_ref[0, :, 0:1]
    d2b = dcol_ref[0, :, 1:2]
    tb = 0.5 * (jnp.log(d2b) - jnp.log(d1b))
    gcol = jax.lax.broadcasted_iota(jnp.int32, (RB, N), 1)
    grow = jax.lax.broadcasted_iota(jnp.int32, (RB, N), 0) + i * RB
    less = (tf < tb).astype(jnp.float32)
    ties = jnp.logical_and(tf == tb, gcol < grow).astype(jnp.float32)
    rank = jnp.sum(less + ties, axis=1, keepdims=True)
    y = jnp.log(jnp.float32(N)) - jnp.log(jnp.float32(N) - rank)
    sxy = jnp.sum(tb * y)
    sxx = jnp.sum(tb * tb)
    first = i == 0
    o1_ref[...] = jnp.where(first, 0.0, o1_ref[...]) + sxy
    o2_ref[...] = jnp.where(first, 0.0, o2_ref[...]) + sxx


def kernel(X):
    d12 = pl.pallas_call(
        _knn2_kernel,
        grid=(B, N // BI, N // BJ),
        in_specs=[
            pl.BlockSpec((1, BI, D), lambda b, i, j: (b, i, 0)),
            pl.BlockSpec((1, BJ, D), lambda b, i, j: (b, j, 0)),
        ],
        out_specs=pl.BlockSpec((1, 2, BI), lambda b, i, j: (b, 0, i)),
        out_shape=jax.ShapeDtypeStruct((B, 2, N), jnp.float32),
        compiler_params=pltpu.CompilerParams(
            dimension_semantics=("parallel", "parallel", "arbitrary"),
        ),
    )(X, X)
    d12_t = jnp.transpose(d12, (0, 2, 1))
    o1, o2 = pl.pallas_call(
        _twonn_kernel,
        grid=(B, N // RB),
        in_specs=[
            pl.BlockSpec((1, 2, N), lambda b, i: (b, 0, 0)),
            pl.BlockSpec((1, RB, 2), lambda b, i: (b, i, 0)),
        ],
        out_specs=[
            pl.BlockSpec((1, 8, 128), lambda b, i: (b, 0, 0)),
            pl.BlockSpec((1, 8, 128), lambda b, i: (b, 0, 0)),
        ],
        out_shape=[
            jax.ShapeDtypeStruct((B, 8, 128), jnp.float32),
            jax.ShapeDtypeStruct((B, 8, 128), jnp.float32),
        ],
        compiler_params=pltpu.CompilerParams(
            dimension_semantics=("parallel", "arbitrary"),
        ),
    )(d12, d12_t)
    return o1[:, 0, 0] / o2[:, 0, 0]


# speedup vs baseline: 55.8991x; 1.0813x over previous
import jax
import jax.numpy as jnp
from jax.experimental import pallas as pl
from jax.experimental.pallas import tpu as pltpu

B = 2
N = 4096
D = 256
BI = 512
BJ = 512
RB = 512
NJ = N // BJ


def _top2_tournament(s):
    r = s.shape[0]
    h = r // 2
    t1 = jnp.minimum(s[:h], s[h:])
    t2 = jnp.maximum(s[:h], s[h:])
    r = h
    while r > 1:
        h = r // 2
        a1, b1 = t1[:h], t1[h:]
        a2, b2 = t2[:h], t2[h:]
        t1, t2 = (
            jnp.minimum(a1, b1),
            jnp.minimum(jnp.maximum(a1, b1), jnp.minimum(a2, b2)),
        )
        r = h
    return t1, t2


def _knn2_kernel(xi_ref, xj_ref, out_ref):
    i = pl.program_id(1)
    j = pl.program_id(2)
    xi = xi_ref[0]
    xj = xj_ref[0]
    sqj = jnp.sum(xj * xj, axis=1)
    dotm2 = jax.lax.dot_general(
        xj * jnp.float32(-2.0), xi, (((1,), (1,)), ((), ())),
        preferred_element_type=jnp.float32,
    )
    s = dotm2 + sqj[:, None]
    inf = jnp.float32(jnp.inf)
    s = jax.lax.cond(
        i == j,
        lambda v: jnp.where(
            jax.lax.broadcasted_iota(jnp.int32, (BJ, BI), 0)
            == jax.lax.broadcasted_iota(jnp.int32, (BJ, BI), 1),
            inf,
            v,
        ),
        lambda v: v,
        s,
    )
    m1, m2 = _top2_tournament(s)
    first = j == 0
    r1 = jnp.where(first, inf, out_ref[0, 0:1, :])
    r2 = jnp.where(first, inf, out_ref[0, 1:2, :])
    n1 = jnp.minimum(r1, m1)
    n2 = jnp.minimum(jnp.maximum(r1, m1), jnp.minimum(r2, m2))

    @pl.when(j == NJ - 1)
    def _finalize():
        sqi = jax.lax.dot_general(
            jnp.ones((1, D), jnp.float32), xi * xi,
            (((1,), (1,)), ((), ())), preferred_element_type=jnp.float32,
        )
        out_ref[0, 0:1, :] = jnp.maximum(n1 + sqi, 0.0)
        out_ref[0, 1:2, :] = jnp.maximum(n2 + sqi, 0.0)

    @pl.when(j != NJ - 1)
    def _carry():
        out_ref[0, 0:1, :] = n1
        out_ref[0, 1:2, :] = n2


def _twonn_kernel(dfull_ref, dcol_ref, o1_ref, o2_ref):
    i = pl.program_id(1)
    d1f = dfull_ref[0, 0:1, :]
    d2f = dfull_ref[0, 1:2, :]
    tf = 0.5 * (jnp.log(d2f) - jnp.log(d1f))
    d1b = dcol_ref[0, :, 0:1]
    d2b = dcol_ref[0, :, 1:2]
    tb = 0.5 * (jnp.log(d2b) - jnp.log(d1b))
    less = (tf < tb).astype(jnp.float32)
    rank = jnp.sum(less, axis=1, keepdims=True)
    y = jnp.log(jnp.float32(N)) - jnp.log(jnp.float32(N) - rank)
    sxy = jnp.sum(tb * y)
    sxx = jnp.sum(tb * tb)
    first = i == 0
    o1_ref[...] = jnp.where(first, 0.0, o1_ref[...]) + sxy
    o2_ref[...] = jnp.where(first, 0.0, o2_ref[...]) + sxx


def kernel(X):
    d12 = pl.pallas_call(
        _knn2_kernel,
        grid=(B, N // BI, N // BJ),
        in_specs=[
            pl.BlockSpec((1, BI, D), lambda b, i, j: (b, i, 0)),
            pl.BlockSpec((1, BJ, D), lambda b, i, j: (b, j, 0)),
        ],
        out_specs=pl.BlockSpec((1, 2, BI), lambda b, i, j: (b, 0, i)),
        out_shape=jax.ShapeDtypeStruct((B, 2, N), jnp.float32),
        compiler_params=pltpu.CompilerParams(
            dimension_semantics=("parallel", "parallel", "arbitrary"),
        ),
    )(X, X)
    d12_t = jnp.transpose(d12, (0, 2, 1))
    o1, o2 = pl.pallas_call(
        _twonn_kernel,
        grid=(B, N // RB),
        in_specs=[
            pl.BlockSpec((1, 2, N), lambda b, i: (b, 0, 0)),
            pl.BlockSpec((1, RB, 2), lambda b, i: (b, i, 0)),
        ],
        out_specs=[
            pl.BlockSpec((1, 8, 128), lambda b, i: (b, 0, 0)),
            pl.BlockSpec((1, 8, 128), lambda b, i: (b, 0, 0)),
        ],
        out_shape=[
            jax.ShapeDtypeStruct((B, 8, 128), jnp.float32),
            jax.ShapeDtypeStruct((B, 8, 128), jnp.float32),
        ],
        compiler_params=pltpu.CompilerParams(
            dimension_semantics=("parallel", "arbitrary"),
        ),
    )(d12, d12_t)
    return o1[:, 0, 0] / o2[:, 0, 0]


# speedup vs baseline: 121.5601x; 2.1746x over previous
import jax
import jax.numpy as jnp
from jax.experimental import pallas as pl
from jax.experimental.pallas import tpu as pltpu

B = 2
N = 4096
D = 256
BI = 512
RB = 512


def _top3_tournament(s):
    r = s.shape[0] // 2
    t1 = jnp.minimum(s[:r], s[r:])
    t2 = jnp.maximum(s[:r], s[r:])
    r //= 2
    a1, b1 = t1[:r], t1[r:]
    a2, b2 = t2[:r], t2[r:]
    mx1 = jnp.maximum(a1, b1)
    mn2 = jnp.minimum(a2, b2)
    t1 = jnp.minimum(a1, b1)
    t3 = jnp.maximum(mx1, mn2)
    t2 = jnp.minimum(mx1, mn2)
    while r > 1:
        r //= 2
        a1, b1 = t1[:r], t1[r:]
        a2, b2 = t2[:r], t2[r:]
        a3, b3 = t3[:r], t3[r:]
        mx1 = jnp.maximum(a1, b1)
        mn2 = jnp.minimum(a2, b2)
        mx2 = jnp.maximum(a2, b2)
        mn3 = jnp.minimum(a3, b3)
        t1 = jnp.minimum(a1, b1)
        t2 = jnp.minimum(mx1, mn2)
        t3 = jnp.minimum(jnp.maximum(mx1, mn2), jnp.minimum(mx2, mn3))
    return t1, t2, t3


def _knn2_kernel(xi_ref, xj_ref, out_ref):
    xi = xi_ref[0]
    xj = xj_ref[0]
    sqjh = 0.5 * jnp.sum(xj * xj, axis=1)
    dot = jax.lax.dot_general(
        xj, xi, (((1,), (1,)), ((), ())), preferred_element_type=jnp.float32
    )
    s = sqjh[:, None] - dot
    _, m2, m3 = _top3_tournament(s)
    sqi = jax.lax.dot_general(
        jnp.ones((1, D), jnp.float32), xi * xi,
        (((1,), (1,)), ((), ())), preferred_element_type=jnp.float32,
    )
    out_ref[0, 0:1, :] = jnp.maximum(2.0 * m2 + sqi, 0.0)
    out_ref[0, 1:2, :] = jnp.maximum(2.0 * m3 + sqi, 0.0)


def _twonn_kernel(dfull_ref, dcol_ref, o1_ref, o2_ref):
    i = pl.program_id(1)
    d1f = dfull_ref[0, 0:1, :]
    d2f = dfull_ref[0, 1:2, :]
    tf = 0.5 * (jnp.log(d2f) - jnp.log(d1f))
    d1b = dcol_ref[0, :, 0:1]
    d2b = dcol_ref[0, :, 1:2]
    tb = 0.5 * (jnp.log(d2b) - jnp.log(d1b))
    less = (tf < tb).astype(jnp.float32)
    rank = jnp.sum(less, axis=1, keepdims=True)
    y = jnp.log(jnp.float32(N)) - jnp.log(jnp.float32(N) - rank)
    sxy = jnp.sum(tb * y)
    sxx = jnp.sum(tb * tb)
    first = i == 0
    o1_ref[...] = jnp.where(first, 0.0, o1_ref[...]) + sxy
    o2_ref[...] = jnp.where(first, 0.0, o2_ref[...]) + sxx


def kernel(X):
    d12 = pl.pallas_call(
        _knn2_kernel,
        grid=(B, N // BI),
        in_specs=[
            pl.BlockSpec((1, BI, D), lambda b, i: (b, i, 0)),
            pl.BlockSpec((1, N, D), lambda b, i: (b, 0, 0)),
        ],
        out_specs=pl.BlockSpec((1, 2, BI), lambda b, i: (b, 0, i)),
        out_shape=jax.ShapeDtypeStruct((B, 2, N), jnp.float32),
        compiler_params=pltpu.CompilerParams(
            dimension_semantics=("parallel", "parallel"),
        ),
    )(X, X)
    d12_t = jnp.transpose(d12, (0, 2, 1))
    o1, o2 = pl.pallas_call(
        _twonn_kernel,
        grid=(B, N // RB),
        in_specs=[
            pl.BlockSpec((1, 2, N), lambda b, i: (b, 0, 0)),
            pl.BlockSpec((1, RB, 2), lambda b, i: (b, i, 0)),
        ],
        out_specs=[
            pl.BlockSpec((1, 8, 128), lambda b, i: (b, 0, 0)),
            pl.BlockSpec((1, 8, 128), lambda b, i: (b, 0, 0)),
        ],
        out_shape=[
            jax.ShapeDtypeStruct((B, 8, 128), jnp.float32),
            jax.ShapeDtypeStruct((B, 8, 128), jnp.float32),
        ],
        compiler_params=pltpu.CompilerParams(
            dimension_semantics=("parallel", "arbitrary"),
        ),
    )(d12, d12_t)
    return o1[:, 0, 0] / o2[:, 0, 0]


# speedup vs baseline: 128.2727x; 1.0552x over previous
import jax
import jax.numpy as jnp
from jax.experimental import pallas as pl
from jax.experimental.pallas import tpu as pltpu

B = 2
N = 4096
D = 256
BI = 512
RB = 512


def _top3_tournament(s):
    r = s.shape[0] // 2
    t1 = jnp.minimum(s[:r], s[r:])
    t2 = jnp.maximum(s[:r], s[r:])
    r //= 2
    a1, b1 = t1[:r], t1[r:]
    a2, b2 = t2[:r], t2[r:]
    mx1 = jnp.maximum(a1, b1)
    mn2 = jnp.minimum(a2, b2)
    t1 = jnp.minimum(a1, b1)
    t3 = jnp.maximum(mx1, mn2)
    t2 = jnp.minimum(mx1, mn2)
    while r > 1:
        r //= 2
        a1, b1 = t1[:r], t1[r:]
        a2, b2 = t2[:r], t2[r:]
        a3, b3 = t3[:r], t3[r:]
        mx1 = jnp.maximum(a1, b1)
        mn2 = jnp.minimum(a2, b2)
        mx2 = jnp.maximum(a2, b2)
        mn3 = jnp.minimum(a3, b3)
        t1 = jnp.minimum(a1, b1)
        t2 = jnp.minimum(mx1, mn2)
        t3 = jnp.minimum(jnp.maximum(mx1, mn2), jnp.minimum(mx2, mn3))
    return t1, t2, t3


def _knn2_kernel(xi_ref, xj_ref, out_ref, sqjh_ref):
    @pl.when(pl.program_id(1) == 0)
    def _norms():
        xj = xj_ref[0]
        sqjh_ref[:, 0] = 0.5 * jnp.sum(xj * xj, axis=1)

    xi = xi_ref[0]
    xj = xj_ref[0]
    dot = jax.lax.dot_general(
        xj, xi, (((1,), (1,)), ((), ())), preferred_element_type=jnp.float32
    )
    s = sqjh_ref[...] - dot
    t1, m2, m3 = _top3_tournament(s)
    out_ref[0, 0:1, :] = 2.0 * (m2 - t1)
    out_ref[0, 1:2, :] = 2.0 * (m3 - t1)


def _twonn_kernel(dfull_ref, o1_ref, o2_ref):
    i = pl.program_id(1)
    d1f = dfull_ref[0, 0:1, :]
    d2f = dfull_ref[0, 1:2, :]
    tf = 0.5 * (jnp.log(d2f) - jnp.log(d1f))
    d1s = dfull_ref[0, 0:1, pl.ds(i * RB, RB)]
    d2s = dfull_ref[0, 1:2, pl.ds(i * RB, RB)]
    tb = jnp.transpose(
        0.5 * (jnp.log(d2s) - jnp.log(d1s))
    )
    less = (tf < tb).astype(jnp.float32)
    rank = jnp.sum(less, axis=1, keepdims=True)
    y = jnp.log(jnp.float32(N)) - jnp.log(jnp.float32(N) - rank)
    sxy = jnp.sum(tb * y)
    sxx = jnp.sum(tb * tb)
    first = i == 0
    o1_ref[...] = jnp.where(first, 0.0, o1_ref[...]) + sxy
    o2_ref[...] = jnp.where(first, 0.0, o2_ref[...]) + sxx


def kernel(X):
    d12 = pl.pallas_call(
        _knn2_kernel,
        grid=(B, N // BI),
        in_specs=[
            pl.BlockSpec((1, BI, D), lambda b, i: (b, i, 0)),
            pl.BlockSpec((1, N, D), lambda b, i: (b, 0, 0)),
        ],
        out_specs=pl.BlockSpec((1, 2, BI), lambda b, i: (b, 0, i)),
        out_shape=jax.ShapeDtypeStruct((B, 2, N), jnp.float32),
        scratch_shapes=[pltpu.VMEM((N, 1), jnp.float32)],
        compiler_params=pltpu.CompilerParams(
            dimension_semantics=("parallel", "arbitrary"),
        ),
    )(X, X)
    o1, o2 = pl.pallas_call(
        _twonn_kernel,
        grid=(B, N // RB),
        in_specs=[
            pl.BlockSpec((1, 2, N), lambda b, i: (b, 0, 0)),
        ],
        out_specs=[
            pl.BlockSpec((1, 8, 128), lambda b, i: (b, 0, 0)),
            pl.BlockSpec((1, 8, 128), lambda b, i: (b, 0, 0)),
        ],
        out_shape=[
            jax.ShapeDtypeStruct((B, 8, 128), jnp.float32),
            jax.ShapeDtypeStruct((B, 8, 128), jnp.float32),
        ],
        compiler_params=pltpu.CompilerParams(
            dimension_semantics=("parallel", "arbitrary"),
        ),
    )(d12)
    return o1[:, 0, 0] / o2[:, 0, 0]
